# baseline (speedup 1.0000x reference)
import functools

import jax
import jax.numpy as jnp
from jax.experimental import pallas as pl
from jax.experimental.pallas import tpu as pltpu

L = 4
K = 1024
D = 32
B = 16384
BETA = 0.01

BB = 512
INV_D = 1.0 / D


def _rvq_kernel(x_ref, cb_ref, idx_ref, p0_ref, p1_ref, p2_ref, p3_ref,
                quant_ref, loss_ref, cn_ref):
    p_refs = (p0_ref, p1_ref, p2_ref, p3_ref)

    @pl.when(pl.program_id(0) == 0)
    def _():
        ones_row = jnp.ones((1, D), dtype=jnp.float32)
        for l in range(L):
            cb = cb_ref[l]
            cn_ref[l] = jax.lax.dot_general(
                ones_row, cb * cb, (((1,), (1,)), ((), ())),
                precision=jax.lax.Precision.HIGHEST,
            )

    residual = x_ref[...]
    quantized = jnp.zeros_like(residual)
    iota = jax.lax.broadcasted_iota(jnp.int32, (residual.shape[0], K), 1)
    for l in range(L):
        cb = cb_ref[l]
        rn = jnp.sum(residual * residual, axis=1, keepdims=True)
        mm = jax.lax.dot_general(
            residual, cb, (((1,), (1,)), ((), ())))
        d = (rn - 2.0 * mm) + cn_ref[l]
        dmin = jnp.min(d, axis=1, keepdims=True)
        idx = jnp.min(jnp.where(d == dmin, iota, K), axis=1, keepdims=True)
        e = jnp.exp(dmin - d)
        p = e / jnp.sum(e, axis=1, keepdims=True)
        p_refs[l][...] = p
        idx_ref[:, pl.ds(l, 1)] = idx
        m = dmin * INV_D
        loss_ref[:, pl.ds(l, 1)] = m + BETA * m
        onehot = (iota == idx).astype(jnp.float32)
        q = jax.lax.dot_general(
            onehot, cb, (((1,), (0,)), ((), ())),
            precision=jax.lax.Precision.HIGHEST,
        )
        quants = residual + (q - residual)
        residual = residual - quants
        quantized = quantized + quants
    quant_ref[...] = quantized


def _rvq_half(x, codebooks, nb):
    nrows = nb * BB
    out_shapes = (
        jax.ShapeDtypeStruct((nrows, L), jnp.int32),
        jax.ShapeDtypeStruct((nrows, K), jnp.float32),
        jax.ShapeDtypeStruct((nrows, K), jnp.float32),
        jax.ShapeDtypeStruct((nrows, K), jnp.float32),
        jax.ShapeDtypeStruct((nrows, K), jnp.float32),
        jax.ShapeDtypeStruct((nrows, D), jnp.float32),
        jax.ShapeDtypeStruct((nrows, L), jnp.float32),
    )
    small = pl.BlockSpec((BB, L), lambda i: (i, 0))
    big = pl.BlockSpec((BB, K), lambda i: (i, 0))
    idx, p0, p1, p2, p3, quantized, losses = pl.pallas_call(
        _rvq_kernel,
        grid=(nb,),
        in_specs=[
            pl.BlockSpec((BB, D), lambda i: (i, 0)),
            pl.BlockSpec((L, K, D), lambda i: (0, 0, 0)),
        ],
        out_specs=(
            small, big, big, big, big,
            pl.BlockSpec((BB, D), lambda i: (i, 0)),
            small,
        ),
        out_shape=out_shapes,
        scratch_shapes=[pltpu.VMEM((L, 1, K), jnp.float32)],
    )(x, codebooks)
    soft_probs = jnp.stack([p0, p1, p2, p3], axis=-1)
    return idx, soft_probs, quantized, losses


@jax.jit
def kernel(x, codebooks):
    half = B // 2
    outs0 = _rvq_half(x[:half], codebooks, half // BB)
    outs1 = _rvq_half(x[half:], codebooks, half // BB)
    outs0 = jax.lax.optimization_barrier(outs0)
    outs1 = jax.lax.optimization_barrier(outs1)
    return tuple(
        jnp.concatenate([a, b], axis=0) for a, b in zip(outs0, outs1))


# speedup vs baseline: 1.5138x; 1.5138x over previous
import functools

import jax
import jax.numpy as jnp
from jax.experimental import pallas as pl
from jax.experimental.pallas import tpu as pltpu

L = 4
K = 1024
D = 32
B = 16384
BETA = 0.01

BB = 512
INV_D = 1.0 / D


def _rvq_kernel(x_ref, cb_ref, idx_ref, p0_ref, p1_ref, p2_ref, p3_ref,
                quant_ref, loss_ref, cn_ref, cb1_ref, cb2_ref, cb3_ref):
    p_refs = (p0_ref, p1_ref, p2_ref, p3_ref)

    @pl.when(pl.program_id(0) == 0)
    def _():
        ones_row = jnp.ones((1, D), dtype=jnp.float32)
        for l in range(L):
            cb = cb_ref[l]
            cn_ref[l] = jax.lax.dot_general(
                ones_row, cb * cb, (((1,), (1,)), ((), ())),
                precision=jax.lax.Precision.HIGHEST,
            )
            m1 = cb.astype(jnp.bfloat16)
            r1 = cb - m1.astype(jnp.float32)
            m2 = r1.astype(jnp.bfloat16)
            r2 = r1 - m2.astype(jnp.float32)
            cb1_ref[l] = m1
            cb2_ref[l] = m2
            cb3_ref[l] = r2.astype(jnp.bfloat16)

    residual = x_ref[...]
    quantized = jnp.zeros_like(residual)
    iota = jax.lax.broadcasted_iota(jnp.int32, (residual.shape[0], K), 1)
    for l in range(L):
        cb = cb_ref[l]
        rn = jnp.sum(residual * residual, axis=1, keepdims=True)
        mm = jax.lax.dot_general(
            residual, cb, (((1,), (1,)), ((), ())))
        d = (rn - 2.0 * mm) + cn_ref[l]
        dmin = jnp.min(d, axis=1, keepdims=True)
        idx = jnp.min(jnp.where(d == dmin, iota, K), axis=1, keepdims=True)
        e = jnp.exp(dmin - d)
        p = e / jnp.sum(e, axis=1, keepdims=True)
        p_refs[l][...] = p
        idx_ref[:, pl.ds(l, 1)] = idx
        m = dmin * INV_D
        loss_ref[:, pl.ds(l, 1)] = m + BETA * m
        onehot = (iota == idx).astype(jnp.bfloat16)
        dims = (((1,), (0,)), ((), ()))
        q = jax.lax.dot_general(
            onehot, cb1_ref[l], dims, preferred_element_type=jnp.float32)
        q = q + jax.lax.dot_general(
            onehot, cb2_ref[l], dims, preferred_element_type=jnp.float32)
        q = q + jax.lax.dot_general(
            onehot, cb3_ref[l], dims, preferred_element_type=jnp.float32)
        quants = residual + (q - residual)
        residual = residual - quants
        quantized = quantized + quants
    quant_ref[...] = quantized


def _rvq_half(x, codebooks, nb):
    nrows = nb * BB
    out_shapes = (
        jax.ShapeDtypeStruct((nrows, L), jnp.int32),
        jax.ShapeDtypeStruct((nrows, K), jnp.float32),
        jax.ShapeDtypeStruct((nrows, K), jnp.float32),
        jax.ShapeDtypeStruct((nrows, K), jnp.float32),
        jax.ShapeDtypeStruct((nrows, K), jnp.float32),
        jax.ShapeDtypeStruct((nrows, D), jnp.float32),
        jax.ShapeDtypeStruct((nrows, L), jnp.float32),
    )
    small = pl.BlockSpec((BB, L), lambda i: (i, 0))
    big = pl.BlockSpec((BB, K), lambda i: (i, 0))
    idx, p0, p1, p2, p3, quantized, losses = pl.pallas_call(
        _rvq_kernel,
        grid=(nb,),
        in_specs=[
            pl.BlockSpec((BB, D), lambda i: (i, 0)),
            pl.BlockSpec((L, K, D), lambda i: (0, 0, 0)),
        ],
        out_specs=(
            small, big, big, big, big,
            pl.BlockSpec((BB, D), lambda i: (i, 0)),
            small,
        ),
        out_shape=out_shapes,
        scratch_shapes=[
            pltpu.VMEM((L, 1, K), jnp.float32),
            pltpu.VMEM((L, K, D), jnp.bfloat16),
            pltpu.VMEM((L, K, D), jnp.bfloat16),
            pltpu.VMEM((L, K, D), jnp.bfloat16),
        ],
    )(x, codebooks)
    soft_probs = jnp.stack([p0, p1, p2, p3], axis=-1)
    return idx, soft_probs, quantized, losses


@jax.jit
def kernel(x, codebooks):
    return _rvq_half(x, codebooks, B // BB)


# speedup vs baseline: 1.6646x; 1.0996x over previous
import functools

import jax
import jax.numpy as jnp
from jax.experimental import pallas as pl
from jax.experimental.pallas import tpu as pltpu

L = 4
K = 1024
D = 32
B = 16384
BETA = 0.01

BB = 256
INV_D = 1.0 / D


def _rvq_kernel(x_ref, cb_ref, idx_ref, p0_ref, p1_ref, p2_ref, p3_ref,
                quant_ref, loss_ref, cn_ref, cb1_ref, cb2_ref, cb3_ref):
    p_refs = (p0_ref, p1_ref, p2_ref, p3_ref)

    @pl.when(pl.program_id(0) == 0)
    def _():
        ones_row = jnp.ones((1, D), dtype=jnp.float32)
        for l in range(L):
            cb = cb_ref[l]
            cn_ref[l] = jax.lax.dot_general(
                ones_row, cb * cb, (((1,), (1,)), ((), ())),
                precision=jax.lax.Precision.HIGHEST,
            )
            m1 = cb.astype(jnp.bfloat16)
            r1 = cb - m1.astype(jnp.float32)
            m2 = r1.astype(jnp.bfloat16)
            r2 = r1 - m2.astype(jnp.float32)
            cb1_ref[l] = m1
            cb2_ref[l] = m2
            cb3_ref[l] = r2.astype(jnp.bfloat16)

    residual = x_ref[...]
    quantized = jnp.zeros_like(residual)
    iota = jax.lax.broadcasted_iota(jnp.int32, (residual.shape[0], K), 1)
    for l in range(L):
        cb = cb_ref[l]
        rn = jnp.sum(residual * residual, axis=1, keepdims=True)
        mm = jax.lax.dot_general(
            residual, cb, (((1,), (1,)), ((), ())))
        d = (rn - 2.0 * mm) + cn_ref[l]
        dmin = jnp.min(d, axis=1, keepdims=True)
        idx = jnp.min(jnp.where(d == dmin, iota, K), axis=1, keepdims=True)
        e = jnp.exp(dmin - d)
        p = e * (1.0 / jnp.sum(e, axis=1, keepdims=True))
        p_refs[l][...] = p
        idx_ref[:, pl.ds(l, 1)] = idx
        m = dmin * INV_D
        loss_ref[:, pl.ds(l, 1)] = m + BETA * m
        onehot = (iota == idx).astype(jnp.bfloat16)
        dims = (((1,), (0,)), ((), ()))
        q = jax.lax.dot_general(
            onehot, cb1_ref[l], dims, preferred_element_type=jnp.float32)
        q = q + jax.lax.dot_general(
            onehot, cb2_ref[l], dims, preferred_element_type=jnp.float32)
        q = q + jax.lax.dot_general(
            onehot, cb3_ref[l], dims, preferred_element_type=jnp.float32)
        quants = residual + (q - residual)
        residual = residual - quants
        quantized = quantized + quants
    quant_ref[...] = quantized


def _rvq_half(x, codebooks, nb):
    nrows = nb * BB
    out_shapes = (
        jax.ShapeDtypeStruct((nrows, L), jnp.int32),
        jax.ShapeDtypeStruct((nrows, K), jnp.float32),
        jax.ShapeDtypeStruct((nrows, K), jnp.float32),
        jax.ShapeDtypeStruct((nrows, K), jnp.float32),
        jax.ShapeDtypeStruct((nrows, K), jnp.float32),
        jax.ShapeDtypeStruct((nrows, D), jnp.float32),
        jax.ShapeDtypeStruct((nrows, L), jnp.float32),
    )
    small = pl.BlockSpec((BB, L), lambda i: (i, 0))
    big = pl.BlockSpec((BB, K), lambda i: (i, 0))
    idx, p0, p1, p2, p3, quantized, losses = pl.pallas_call(
        _rvq_kernel,
        grid=(nb,),
        in_specs=[
            pl.BlockSpec((BB, D), lambda i: (i, 0)),
            pl.BlockSpec((L, K, D), lambda i: (0, 0, 0)),
        ],
        out_specs=(
            small, big, big, big, big,
            pl.BlockSpec((BB, D), lambda i: (i, 0)),
            small,
        ),
        out_shape=out_shapes,
        scratch_shapes=[
            pltpu.VMEM((L, 1, K), jnp.float32),
            pltpu.VMEM((L, K, D), jnp.bfloat16),
            pltpu.VMEM((L, K, D), jnp.bfloat16),
            pltpu.VMEM((L, K, D), jnp.bfloat16),
        ],
    )(x, codebooks)
    soft_probs = jnp.stack([p0, p1, p2, p3], axis=-1)
    return idx, soft_probs, quantized, losses


@jax.jit
def kernel(x, codebooks):
    return _rvq_half(x, codebooks, B // BB)


# speedup vs baseline: 1.6729x; 1.0050x over previous
import functools

import jax
import jax.numpy as jnp
from jax.experimental import pallas as pl
from jax.experimental.pallas import tpu as pltpu

L = 4
K = 1024
D = 32
B = 16384
BETA = 0.01

BB = 512
NSUB = 2
INV_D = 1.0 / D


def _rvq_kernel(x_ref, cb_ref, idx_ref, p0_ref, p1_ref, p2_ref, p3_ref,
                quant_ref, loss_ref, cn_ref, cb1_ref, cb2_ref, cb3_ref):
    p_refs = (p0_ref, p1_ref, p2_ref, p3_ref)

    @pl.when(pl.program_id(0) == 0)
    def _():
        ones_row = jnp.ones((1, D), dtype=jnp.float32)
        for l in range(L):
            cb = cb_ref[l]
            cn_ref[l] = jax.lax.dot_general(
                ones_row, cb * cb, (((1,), (1,)), ((), ())),
                precision=jax.lax.Precision.HIGHEST,
            )
            m1 = cb.astype(jnp.bfloat16)
            r1 = cb - m1.astype(jnp.float32)
            m2 = r1.astype(jnp.bfloat16)
            r2 = r1 - m2.astype(jnp.float32)
            cb1_ref[l] = m1
            cb2_ref[l] = m2
            cb3_ref[l] = r2.astype(jnp.bfloat16)

    hb = BB // NSUB
    iota = jax.lax.broadcasted_iota(jnp.int32, (hb, K), 1)
    for h in range(NSUB):
        rows = pl.ds(h * hb, hb)
        residual = x_ref[rows, :]
        quantized = jnp.zeros_like(residual)
        for l in range(L):
            rn = jnp.sum(residual * residual, axis=1, keepdims=True)
            mm = jax.lax.dot_general(
                residual, cb_ref[l], (((1,), (1,)), ((), ())))
            d = (rn - 2.0 * mm) + cn_ref[l]
            dmin = jnp.min(d, axis=1, keepdims=True)
            idx = jnp.min(jnp.where(d == dmin, iota, K), axis=1,
                          keepdims=True)
            e = jnp.exp(dmin - d)
            p = e * (1.0 / jnp.sum(e, axis=1, keepdims=True))
            p_refs[l][rows, :] = p
            idx_ref[rows, pl.ds(l, 1)] = idx
            m = dmin * INV_D
            loss_ref[rows, pl.ds(l, 1)] = m + BETA * m
            onehot = (iota == idx).astype(jnp.bfloat16)
            dims = (((1,), (0,)), ((), ()))
            q = jax.lax.dot_general(
                onehot, cb1_ref[l], dims,
                preferred_element_type=jnp.float32)
            q = q + jax.lax.dot_general(
                onehot, cb2_ref[l], dims,
                preferred_element_type=jnp.float32)
            q = q + jax.lax.dot_general(
                onehot, cb3_ref[l], dims,
                preferred_element_type=jnp.float32)
            quants = residual + (q - residual)
            residual = residual - quants
            quantized = quantized + quants
        quant_ref[rows, :] = quantized


def _rvq_half(x, codebooks, nb):
    nrows = nb * BB
    out_shapes = (
        jax.ShapeDtypeStruct((nrows, L), jnp.int32),
        jax.ShapeDtypeStruct((nrows, K), jnp.float32),
        jax.ShapeDtypeStruct((nrows, K), jnp.float32),
        jax.ShapeDtypeStruct((nrows, K), jnp.float32),
        jax.ShapeDtypeStruct((nrows, K), jnp.float32),
        jax.ShapeDtypeStruct((nrows, D), jnp.float32),
        jax.ShapeDtypeStruct((nrows, L), jnp.float32),
    )
    small = pl.BlockSpec((BB, L), lambda i: (i, 0))
    big = pl.BlockSpec((BB, K), lambda i: (i, 0))
    idx, p0, p1, p2, p3, quantized, losses = pl.pallas_call(
        _rvq_kernel,
        grid=(nb,),
        in_specs=[
            pl.BlockSpec((BB, D), lambda i: (i, 0)),
            pl.BlockSpec((L, K, D), lambda i: (0, 0, 0)),
        ],
        out_specs=(
            small, big, big, big, big,
            pl.BlockSpec((BB, D), lambda i: (i, 0)),
            small,
        ),
        out_shape=out_shapes,
        scratch_shapes=[
            pltpu.VMEM((L, 1, K), jnp.float32),
            pltpu.VMEM((L, K, D), jnp.bfloat16),
            pltpu.VMEM((L, K, D), jnp.bfloat16),
            pltpu.VMEM((L, K, D), jnp.bfloat16),
        ],
    )(x, codebooks)
    soft_probs = jnp.stack([p0, p1, p2, p3], axis=-1)
    return idx, soft_probs, quantized, losses


@jax.jit
def kernel(x, codebooks):
    return _rvq_half(x, codebooks, B // BB)
